# baseline (speedup 1.0000x reference)
import jax
import jax.numpy as jnp
from jax.experimental import pallas as pl
from jax.experimental.pallas import tpu as pltpu

_BLOCK_ROWS = 512


def _clip_kernel(x_ref, o_ref):
    o_ref[...] = jnp.clip(x_ref[...], 0.0, 1.0)


def kernel(x):
    n_rows, n_cols = x.shape
    grid = (n_rows // _BLOCK_ROWS,)
    return pl.pallas_call(
        _clip_kernel,
        grid=grid,
        in_specs=[pl.BlockSpec((_BLOCK_ROWS, n_cols), lambda i: (i, 0))],
        out_specs=pl.BlockSpec((_BLOCK_ROWS, n_cols), lambda i: (i, 0)),
        out_shape=jax.ShapeDtypeStruct((n_rows, n_cols), x.dtype),
        compiler_params=pltpu.CompilerParams(
            dimension_semantics=("arbitrary",),
        ),
    )(x)


# speedup vs baseline: 1.0109x; 1.0109x over previous
import jax
import jax.numpy as jnp
from jax.experimental import pallas as pl
from jax.experimental.pallas import tpu as pltpu

_BLOCK_ROWS = 1024


def _clip_kernel(x_ref, o_ref):
    o_ref[...] = jnp.clip(x_ref[...], 0.0, 1.0)


def kernel(x):
    n_rows, n_cols = x.shape
    grid = (n_rows // _BLOCK_ROWS,)
    return pl.pallas_call(
        _clip_kernel,
        grid=grid,
        in_specs=[pl.BlockSpec((_BLOCK_ROWS, n_cols), lambda i: (i, 0))],
        out_specs=pl.BlockSpec((_BLOCK_ROWS, n_cols), lambda i: (i, 0)),
        out_shape=jax.ShapeDtypeStruct((n_rows, n_cols), x.dtype),
        compiler_params=pltpu.CompilerParams(
            dimension_semantics=("arbitrary",),
        ),
    )(x)
